# baseline (speedup 1.0000x reference)
import functools
import jax
import jax.numpy as jnp
from jax import lax
from jax.experimental import pallas as pl
from jax.experimental.pallas import tpu as pltpu
from jax.experimental.pallas import tpu_sc as plsc

N = 10000
E = 320000
D = 128

NC = 2
NS = 16
NW = NC * NS

E_PER_TILE = E // NW
SCAT_CH = 80
SCAT_ITERS = E_PER_TILE // SCAT_CH
GATH_CH = 128
GATH_CHUNKS = E // GATH_CH
HIGH = jax.lax.Precision.HIGHEST
N_PAD = 10240
N_PER_TILE = N_PAD // NS


def _nf2_body(nf_ref, w_ref, b_ref, o_ref):
    x = jnp.dot(nf_ref[...], w_ref[...], preferred_element_type=jnp.float32,
                precision=HIGH)
    o_ref[...] = jnp.maximum(x + b_ref[...][None, :], 0.0)


def _nf2(nf, W_node, bias_n):
    blk = 2048
    return pl.pallas_call(
        _nf2_body,
        grid=(N_PAD // blk,),
        in_specs=[
            pl.BlockSpec((blk, D), lambda i: (i, 0)),
            pl.BlockSpec((D, D), lambda i: (0, 0)),
            pl.BlockSpec((D,), lambda i: (0,)),
        ],
        out_specs=pl.BlockSpec((blk, D), lambda i: (i, 0)),
        out_shape=jax.ShapeDtypeStruct((N_PAD, D), jnp.float32),
    )(nf, W_node, bias_n)


def _pack_bf16_pairs(x):
    bits = jax.lax.bitcast_convert_type(x, jnp.uint32)
    rne = ((bits >> 16) & 1) + 0x7FFF
    b16 = (bits + rne) >> 16
    word = b16[:, : D // 2] | (b16[:, D // 2:] << 16)
    return jax.lax.bitcast_convert_type(word, jnp.float32)


def _m_body(ef_ref, we_ref, wd1p_ref, bv_ref, o_ref):
    wc = jnp.dot(we_ref[...], wd1p_ref[...].T,
                 preferred_element_type=jnp.float32, precision=HIGH)
    x = jnp.dot(ef_ref[...], wc, preferred_element_type=jnp.float32)
    o_ref[...] = _pack_bf16_pairs(x + bv_ref[...][None, :])


def _m_edges(ef, W_edge, Wd1p, bvec_p):
    blk = 2560
    return pl.pallas_call(
        _m_body,
        grid=(E // blk,),
        in_specs=[
            pl.BlockSpec((blk, D), lambda i: (i, 0)),
            pl.BlockSpec((D, D), lambda i: (0, 0)),
            pl.BlockSpec((D, D), lambda i: (0, 0)),
            pl.BlockSpec((D,), lambda i: (0,)),
        ],
        out_specs=pl.BlockSpec((blk, D // 2), lambda i: (i, 0)),
        out_shape=jax.ShapeDtypeStruct((E, D // 2), jnp.float32),
    )(ef, W_edge, Wd1p, bvec_p)


def _tables_body(sp_ref, dp_ref, nf2_ref, we_ref, wd1p_ref, wd2p_ref,
                 ab_ref, bh_ref):
    i = pl.program_id(0)
    blk = ab_ref.shape[0]
    S = sp_ref[0] + sp_ref[1]
    deg = dp_ref[0, pl.ds(i * blk, blk)] + dp_ref[1, pl.ds(i * blk, blk)]
    nb = jnp.dot(S, we_ref[...], preferred_element_type=jnp.float32,
                 precision=HIGH)
    nb = nb / jnp.maximum(deg, 1.0)[:, None]
    bh = 0.5 * jnp.dot(nf2_ref[...], wd2p_ref[...].T,
                       preferred_element_type=jnp.float32, precision=HIGH)
    ab = jnp.dot(nb, wd1p_ref[...].T,
                 preferred_element_type=jnp.float32, precision=HIGH) + bh
    ab_ref[...] = ab
    bh_ref[...] = bh


def _tables(S_part, deg_part, nf2, W_edge, Wd1p, Wd2p):
    blk = 2048
    return pl.pallas_call(
        _tables_body,
        grid=(N_PAD // blk,),
        in_specs=[
            pl.BlockSpec((NC, blk, D), lambda i: (0, i, 0)),
            pl.BlockSpec((NC, N_PAD), lambda i: (0, 0)),
            pl.BlockSpec((blk, D), lambda i: (i, 0)),
            pl.BlockSpec((D, D), lambda i: (0, 0)),
            pl.BlockSpec((D, D), lambda i: (0, 0)),
            pl.BlockSpec((D, D), lambda i: (0, 0)),
        ],
        out_specs=[
            pl.BlockSpec((blk, D), lambda i: (i, 0)),
            pl.BlockSpec((blk, D), lambda i: (i, 0)),
        ],
        out_shape=[
            jax.ShapeDtypeStruct((N_PAD, D), jnp.float32),
            jax.ShapeDtypeStruct((N_PAD, D), jnp.float32),
        ],
    )(S_part, deg_part, nf2, W_edge, Wd1p, Wd2p)


def _scatter_tec(ef_hbm, dst_hbm, zS_hbm, zdeg_hbm, S_out, deg_out,
                 S_acc, deg_acc, ef_buf, idx_buf, ones_buf,
                 sem_li, sem_le, sem_se, sem_sd):
    c = lax.axis_index("c")
    s = lax.axis_index("s")

    pltpu.sync_copy(zS_hbm.at[pl.ds(s * N_PER_TILE, N_PER_TILE)],
                    S_acc.at[pl.ds(s * N_PER_TILE, N_PER_TILE)])

    @pl.when(s == 0)
    def _():
        pltpu.sync_copy(zdeg_hbm, deg_acc)

    for j in range(SCAT_CH // 16):
        ones_buf[pl.ds(j * 16, 16)] = jnp.ones((16,), jnp.float32)

    plsc.subcore_barrier()

    base = (c * NS + s) * E_PER_TILE

    def issue_load(i, slot):
        off = base + i * SCAT_CH
        pltpu.async_copy(dst_hbm.at[pl.ds(off, SCAT_CH)], idx_buf.at[slot],
                         sem_li.at[slot])
        pltpu.async_copy(ef_hbm.at[pl.ds(off, SCAT_CH)], ef_buf.at[slot],
                         sem_le.at[slot])

    issue_load(0, 0)

    def substep(i, sp, sq):
        off = base + i * SCAT_CH

        @pl.when(i >= 1)
        def _():
            pltpu.make_async_copy(ef_buf.at[sq], S_acc.at[idx_buf.at[sq]],
                                  sem_se.at[sq]).wait()
            pltpu.make_async_copy(ones_buf, deg_acc.at[idx_buf.at[sq]],
                                  sem_sd.at[sq]).wait()

        @pl.when(i + 1 < SCAT_ITERS)
        def _():
            issue_load(i + 1, sq)

        @pl.when(i < SCAT_ITERS)
        def _():
            pltpu.make_async_copy(dst_hbm.at[pl.ds(off, SCAT_CH)],
                                  idx_buf.at[sp], sem_li.at[sp]).wait()
            pltpu.make_async_copy(ef_hbm.at[pl.ds(off, SCAT_CH)],
                                  ef_buf.at[sp], sem_le.at[sp]).wait()
            pltpu.async_copy(ef_buf.at[sp], S_acc.at[idx_buf.at[sp]],
                             sem_se.at[sp], add=True)
            pltpu.async_copy(ones_buf, deg_acc.at[idx_buf.at[sp]],
                             sem_sd.at[sp], add=True)

    def step(g, carry):
        substep(2 * g, 0, 1)
        substep(2 * g + 1, 1, 0)
        return carry

    lax.fori_loop(0, (SCAT_ITERS + 1) // 2, step, 0)

    plsc.subcore_barrier()

    pltpu.sync_copy(S_acc.at[pl.ds(s * N_PER_TILE, N_PER_TILE)],
                    S_out.at[c].at[pl.ds(s * N_PER_TILE, N_PER_TILE)])

    @pl.when(s == 0)
    def _():
        pltpu.sync_copy(deg_acc, deg_out.at[c])


def _scatter(ef, dst, zS, zdeg):
    mesh = plsc.VectorSubcoreMesh(core_axis_name="c", subcore_axis_name="s")
    return pl.kernel(
        _scatter_tec,
        out_type=[
            jax.ShapeDtypeStruct((NC, N_PAD, D), jnp.float32),
            jax.ShapeDtypeStruct((NC, N_PAD), jnp.float32),
        ],
        mesh=mesh,
        scratch_types=[
            pltpu.VMEM_SHARED((N_PAD, D), jnp.float32),
            pltpu.VMEM_SHARED((N_PAD,), jnp.float32),
            pltpu.VMEM((2, SCAT_CH, D), jnp.float32),
            pltpu.VMEM((2, SCAT_CH), jnp.int32),
            pltpu.VMEM((SCAT_CH,), jnp.float32),
            pltpu.SemaphoreType.DMA((2,)),
            pltpu.SemaphoreType.DMA((2,)),
            pltpu.SemaphoreType.DMA((2,)),
            pltpu.SemaphoreType.DMA((2,)),
        ],
    )(ef, dst, zS, zdeg)


def _final_tec(m_hbm, ab_hbm, bh_hbm, idx_hbm, out_hbm,
               buf_m, buf_a, buf_b, ibuf,
               sem_i, sem_a, sem_b, sem_m, sem_o):
    c = lax.axis_index("c")
    s = lax.axis_index("s")
    wid = c * NS + s
    niter = jnp.where(wid < GATH_CHUNKS - (GATH_CHUNKS // NW) * NW,
                      GATH_CHUNKS // NW + 1, GATH_CHUNKS // NW)

    def issue_idx(i, slot):
        pltpu.async_copy(idx_hbm.at[wid + i * NW], ibuf.at[slot],
                         sem_i.at[slot])

    issue_idx(0, 0)

    def substep(i, sp, sq):
        off = (wid + i * NW) * GATH_CH
        offq = (wid + (i - 1) * NW) * GATH_CH

        @pl.when(i < niter)
        def _():
            pltpu.make_async_copy(idx_hbm.at[wid + i * NW], ibuf.at[sp],
                                  sem_i.at[sp]).wait()

        @pl.when(jnp.logical_and(i >= 1, i <= niter))
        def _():
            pltpu.make_async_copy(ab_hbm.at[ibuf.at[sq].at[0]], buf_a.at[sq],
                                  sem_a.at[sq]).wait()
            pltpu.make_async_copy(bh_hbm.at[ibuf.at[sq].at[1]], buf_b.at[sq],
                                  sem_b.at[sq]).wait()
            pltpu.make_async_copy(m_hbm.at[pl.ds(offq, GATH_CH)],
                                  buf_m.at[sq], sem_m.at[sq]).wait()

        @pl.when(i + 1 < niter)
        def _():
            issue_idx(i + 1, sq)

        @pl.when(jnp.logical_and(i >= 2, i <= niter))
        def _():
            offp = (wid + (i - 2) * NW) * GATH_CH
            pltpu.make_async_copy(buf_a.at[sp],
                                  out_hbm.at[pl.ds(offp, GATH_CH)],
                                  sem_o.at[sp]).wait()

        @pl.when(i < niter)
        def _():
            pltpu.async_copy(ab_hbm.at[ibuf.at[sp].at[0]], buf_a.at[sp],
                             sem_a.at[sp])
            pltpu.async_copy(bh_hbm.at[ibuf.at[sp].at[1]], buf_b.at[sp],
                             sem_b.at[sp])
            pltpu.async_copy(m_hbm.at[pl.ds(off, GATH_CH)], buf_m.at[sp],
                             sem_m.at[sp])

        @pl.when(jnp.logical_and(i >= 1, i <= niter))
        def _():
            def unpk(v):
                bits = jax.lax.bitcast_convert_type(v, jnp.uint32)
                lo = jax.lax.bitcast_convert_type(bits << 16, jnp.float32)
                hi = jax.lax.bitcast_convert_type(
                    bits & jnp.uint32(0xFFFF0000), jnp.float32)
                return lo, hi

            def row(r, rc):
                for k in range(D // 32):
                    m0, m1 = unpk(buf_m[sq, r, pl.ds(k * 16, 16)])
                    a0 = buf_a[sq, r, pl.ds(k * 32, 16)]
                    a1 = buf_a[sq, r, pl.ds(k * 32 + 16, 16)]
                    b0 = buf_b[sq, r, pl.ds(k * 32, 16)]
                    b1 = buf_b[sq, r, pl.ds(k * 32 + 16, 16)]
                    buf_a[sq, r, pl.ds(k * 32, 16)] = jnp.maximum(
                        m0 + a0 + b0, 0.0)
                    buf_a[sq, r, pl.ds(k * 32 + 16, 16)] = jnp.maximum(
                        m1 + a1 + b1, 0.0)
                return rc

            lax.fori_loop(0, GATH_CH, row, 0)
            pltpu.async_copy(buf_a.at[sq], out_hbm.at[pl.ds(offq, GATH_CH)],
                             sem_o.at[sq])

    def step(g, carry):
        substep(2 * g, 0, 1)
        substep(2 * g + 1, 1, 0)
        return carry

    lax.fori_loop(0, (GATH_CHUNKS // NW + 2) // 2, step, 0)

    last = niter - 1
    lastslot = lax.rem(last, 2)
    pltpu.make_async_copy(buf_a.at[lastslot],
                          out_hbm.at[pl.ds((wid + last * NW) * GATH_CH,
                                           GATH_CH)],
                          sem_o.at[lastslot]).wait()


def _final(M, AB, Bh, idx_packed):
    mesh = plsc.VectorSubcoreMesh(core_axis_name="c", subcore_axis_name="s")
    return pl.kernel(
        _final_tec,
        out_type=jax.ShapeDtypeStruct((E, D), jnp.float32),
        mesh=mesh,
        scratch_types=[
            pltpu.VMEM((2, GATH_CH, D // 2), jnp.float32),
            pltpu.VMEM((2, GATH_CH, D), jnp.float32),
            pltpu.VMEM((2, GATH_CH, D), jnp.float32),
            pltpu.VMEM((2, 2, GATH_CH), jnp.int32),
            pltpu.SemaphoreType.DMA((2,)),
            pltpu.SemaphoreType.DMA((2,)),
            pltpu.SemaphoreType.DMA((2,)),
            pltpu.SemaphoreType.DMA((2,)),
            pltpu.SemaphoreType.DMA((2,)),
        ],
    )(M, AB, Bh, idx_packed)


@jax.jit
def kernel(nf, ef, edge_index, W_node, W_edge, bias_n, bias_e, W_dense, b_dense):
    src = edge_index[0]
    dst = edge_index[1]
    zS = jnp.zeros((N_PAD, D), jnp.float32)
    zdeg = jnp.zeros((N_PAD,), jnp.float32)
    nf_pad = jnp.concatenate([nf, jnp.zeros((N_PAD - N, D), jnp.float32)], axis=0)

    idx_packed = jnp.stack([dst.reshape(GATH_CHUNKS, GATH_CH),
                            src.reshape(GATH_CHUNKS, GATH_CH)], axis=1)

    v_ = jnp.arange(D)
    perm = 32 * ((v_ % (D // 2)) // 16) + 16 * (v_ // (D // 2)) + (v_ % 16)
    Wd1p = W_dense[perm, :D]
    bvec_p = (b_dense + bias_e)[perm]

    nf2p = _nf2(nf_pad, W_node, bias_n)
    S_part, deg_part = _scatter(ef, dst, zS, zdeg)
    M = _m_edges(ef, W_edge, Wd1p, bvec_p)
    AB, Bh = _tables(S_part, deg_part, nf2p, W_edge, W_dense[:, :D],
                     W_dense[:, D:])
    ef2 = _final(M, AB, Bh, idx_packed)
    return (nf2p[:N], ef2)


# speedup vs baseline: 1.3370x; 1.3370x over previous
import functools
import jax
import jax.numpy as jnp
from jax import lax
from jax.experimental import pallas as pl
from jax.experimental.pallas import tpu as pltpu
from jax.experimental.pallas import tpu_sc as plsc

N = 10000
E = 320000
D = 128

NC = 2
NS = 16
NW = NC * NS

E_PER_TILE = E // NW
SCAT_CH = 128
SCAT_CHUNKS = E // SCAT_CH
GATH_CH = 128
GATH_CHUNKS = E // GATH_CH
HIGH = jax.lax.Precision.HIGHEST
N_PAD = 10240
N_PER_TILE = N_PAD // NS


def _nf2_body(nf_ref, w_ref, b_ref, o_ref):
    x = jnp.dot(nf_ref[...], w_ref[...], preferred_element_type=jnp.float32,
                precision=HIGH)
    o_ref[...] = jnp.maximum(x + b_ref[...][None, :], 0.0)


def _nf2(nf, W_node, bias_n):
    blk = 2048
    return pl.pallas_call(
        _nf2_body,
        grid=(N_PAD // blk,),
        in_specs=[
            pl.BlockSpec((blk, D), lambda i: (i, 0)),
            pl.BlockSpec((D, D), lambda i: (0, 0)),
            pl.BlockSpec((D,), lambda i: (0,)),
        ],
        out_specs=pl.BlockSpec((blk, D), lambda i: (i, 0)),
        out_shape=jax.ShapeDtypeStruct((N_PAD, D), jnp.float32),
    )(nf, W_node, bias_n)


def _m_body(ef_ref, we_ref, wd_ref, bv_ref, o_ref):
    wc = jnp.dot(we_ref[...], wd_ref[...][:, :D].T,
                 preferred_element_type=jnp.float32, precision=HIGH)
    x = jnp.dot(ef_ref[...], wc, preferred_element_type=jnp.float32)
    o_ref[...] = x + bv_ref[...][None, :]


def _m_edges(ef, W_edge, W_dense, bvec):
    blk = 2560
    return pl.pallas_call(
        _m_body,
        grid=(E // blk,),
        in_specs=[
            pl.BlockSpec((blk, D), lambda i: (i, 0)),
            pl.BlockSpec((D, D), lambda i: (0, 0)),
            pl.BlockSpec((D, 2 * D), lambda i: (0, 0)),
            pl.BlockSpec((D,), lambda i: (0,)),
        ],
        out_specs=pl.BlockSpec((blk, D), lambda i: (i, 0)),
        out_shape=jax.ShapeDtypeStruct((E, D), jnp.float32),
    )(ef, W_edge, W_dense, bvec)


def _tables_body(sp_ref, dp_ref, nf2_ref, we_ref, wd_ref, ab_ref, bh_ref):
    i = pl.program_id(0)
    blk = ab_ref.shape[0]
    S = sp_ref[0] + sp_ref[1]
    deg = dp_ref[0, pl.ds(i * blk, blk)] + dp_ref[1, pl.ds(i * blk, blk)]
    nb = jnp.dot(S, we_ref[...], preferred_element_type=jnp.float32,
                 precision=HIGH)
    nb = nb / jnp.maximum(deg, 1.0)[:, None]
    bh = 0.5 * jnp.dot(nf2_ref[...], wd_ref[...][:, D:].T,
                       preferred_element_type=jnp.float32, precision=HIGH)
    ab = jnp.dot(nb, wd_ref[...][:, :D].T,
                 preferred_element_type=jnp.float32, precision=HIGH) + bh
    ab_ref[...] = ab
    bh_ref[...] = bh


def _tables(S_part, deg_part, nf2, W_edge, W_dense):
    blk = 2048
    return pl.pallas_call(
        _tables_body,
        grid=(N_PAD // blk,),
        in_specs=[
            pl.BlockSpec((NC, blk, D), lambda i: (0, i, 0)),
            pl.BlockSpec((NC, N_PAD), lambda i: (0, 0)),
            pl.BlockSpec((blk, D), lambda i: (i, 0)),
            pl.BlockSpec((D, D), lambda i: (0, 0)),
            pl.BlockSpec((D, 2 * D), lambda i: (0, 0)),
        ],
        out_specs=[
            pl.BlockSpec((blk, D), lambda i: (i, 0)),
            pl.BlockSpec((blk, D), lambda i: (i, 0)),
        ],
        out_shape=[
            jax.ShapeDtypeStruct((N_PAD, D), jnp.float32),
            jax.ShapeDtypeStruct((N_PAD, D), jnp.float32),
        ],
    )(S_part, deg_part, nf2, W_edge, W_dense)


def _scatter_tec(ef_hbm, dst_hbm, zS_hbm, zdeg_hbm, S_out, deg_out,
                 S_acc, deg_acc, ef_buf, idx_buf, ones_buf,
                 sem_li, sem_le, sem_se, sem_sd):
    c = lax.axis_index("c")
    s = lax.axis_index("s")

    pltpu.sync_copy(zS_hbm.at[pl.ds(s * N_PER_TILE, N_PER_TILE)],
                    S_acc.at[pl.ds(s * N_PER_TILE, N_PER_TILE)])

    @pl.when(s == 0)
    def _():
        pltpu.sync_copy(zdeg_hbm, deg_acc)

    for j in range(SCAT_CH // 16):
        ones_buf[pl.ds(j * 16, 16)] = jnp.ones((16,), jnp.float32)

    plsc.subcore_barrier()

    wid = c * NS + s
    niter = jnp.where(wid < SCAT_CHUNKS - (SCAT_CHUNKS // NW) * NW,
                      SCAT_CHUNKS // NW + 1, SCAT_CHUNKS // NW)

    def issue_load(i, slot):
        off = (wid + i * NW) * SCAT_CH
        pltpu.async_copy(dst_hbm.at[pl.ds(off, SCAT_CH)], idx_buf.at[slot],
                         sem_li.at[slot])
        pltpu.async_copy(ef_hbm.at[pl.ds(off, SCAT_CH)], ef_buf.at[slot],
                         sem_le.at[slot])

    issue_load(0, 0)

    def substep(i, sp, sq):
        off = (wid + i * NW) * SCAT_CH

        @pl.when(jnp.logical_and(i >= 1, i <= niter))
        def _():
            pltpu.make_async_copy(ef_buf.at[sq], S_acc.at[idx_buf.at[sq]],
                                  sem_se.at[sq]).wait()
            pltpu.make_async_copy(ones_buf, deg_acc.at[idx_buf.at[sq]],
                                  sem_sd.at[sq]).wait()

        @pl.when(i + 1 < niter)
        def _():
            issue_load(i + 1, sq)

        @pl.when(i < niter)
        def _():
            pltpu.make_async_copy(dst_hbm.at[pl.ds(off, SCAT_CH)],
                                  idx_buf.at[sp], sem_li.at[sp]).wait()
            pltpu.make_async_copy(ef_hbm.at[pl.ds(off, SCAT_CH)],
                                  ef_buf.at[sp], sem_le.at[sp]).wait()
            pltpu.async_copy(ef_buf.at[sp], S_acc.at[idx_buf.at[sp]],
                             sem_se.at[sp], add=True)
            pltpu.async_copy(ones_buf, deg_acc.at[idx_buf.at[sp]],
                             sem_sd.at[sp], add=True)

    def step(g, carry):
        substep(2 * g, 0, 1)
        substep(2 * g + 1, 1, 0)
        return carry

    lax.fori_loop(0, (SCAT_CHUNKS // NW + 2) // 2, step, 0)

    plsc.subcore_barrier()

    pltpu.sync_copy(S_acc.at[pl.ds(s * N_PER_TILE, N_PER_TILE)],
                    S_out.at[c].at[pl.ds(s * N_PER_TILE, N_PER_TILE)])

    @pl.when(s == 0)
    def _():
        pltpu.sync_copy(deg_acc, deg_out.at[c])


def _scatter(ef, dst, zS, zdeg):
    mesh = plsc.VectorSubcoreMesh(core_axis_name="c", subcore_axis_name="s")
    return pl.kernel(
        _scatter_tec,
        out_type=[
            jax.ShapeDtypeStruct((NC, N_PAD, D), jnp.float32),
            jax.ShapeDtypeStruct((NC, N_PAD), jnp.float32),
        ],
        mesh=mesh,
        scratch_types=[
            pltpu.VMEM_SHARED((N_PAD, D), jnp.float32),
            pltpu.VMEM_SHARED((N_PAD,), jnp.float32),
            pltpu.VMEM((2, SCAT_CH, D), jnp.float32),
            pltpu.VMEM((2, SCAT_CH), jnp.int32),
            pltpu.VMEM((SCAT_CH,), jnp.float32),
            pltpu.SemaphoreType.DMA((2,)),
            pltpu.SemaphoreType.DMA((2,)),
            pltpu.SemaphoreType.DMA((2,)),
            pltpu.SemaphoreType.DMA((2,)),
        ],
    )(ef, dst, zS, zdeg)


def _final_tec(m_hbm, ab_hbm, bh_hbm, idx_hbm, out_hbm,
               buf_m, buf_a, buf_b, ibuf,
               sem_i, sem_a, sem_b, sem_m, sem_o):
    c = lax.axis_index("c")
    s = lax.axis_index("s")
    wid = c * NS + s
    niter = jnp.where(wid < GATH_CHUNKS - (GATH_CHUNKS // NW) * NW,
                      GATH_CHUNKS // NW + 1, GATH_CHUNKS // NW)

    def issue_idx(i, slot):
        pltpu.async_copy(idx_hbm.at[wid + i * NW], ibuf.at[slot],
                         sem_i.at[slot])

    issue_idx(0, 0)

    def substep(i, sp, sq):
        off = (wid + i * NW) * GATH_CH
        offq = (wid + (i - 1) * NW) * GATH_CH

        @pl.when(i < niter)
        def _():
            pltpu.make_async_copy(idx_hbm.at[wid + i * NW], ibuf.at[sp],
                                  sem_i.at[sp]).wait()

        @pl.when(jnp.logical_and(i >= 1, i <= niter))
        def _():
            pltpu.make_async_copy(ab_hbm.at[ibuf.at[sq].at[0]], buf_a.at[sq],
                                  sem_a.at[sq]).wait()
            pltpu.make_async_copy(bh_hbm.at[ibuf.at[sq].at[1]], buf_b.at[sq],
                                  sem_b.at[sq]).wait()
            pltpu.make_async_copy(m_hbm.at[pl.ds(offq, GATH_CH)],
                                  buf_m.at[sq], sem_m.at[sq]).wait()

        @pl.when(i + 1 < niter)
        def _():
            issue_idx(i + 1, sq)

        @pl.when(jnp.logical_and(i >= 2, i <= niter))
        def _():
            offp = (wid + (i - 2) * NW) * GATH_CH
            pltpu.make_async_copy(buf_m.at[sp],
                                  out_hbm.at[pl.ds(offp, GATH_CH)],
                                  sem_o.at[sp]).wait()

        @pl.when(i < niter)
        def _():
            pltpu.async_copy(ab_hbm.at[ibuf.at[sp].at[0]], buf_a.at[sp],
                             sem_a.at[sp])
            pltpu.async_copy(bh_hbm.at[ibuf.at[sp].at[1]], buf_b.at[sp],
                             sem_b.at[sp])
            pltpu.async_copy(m_hbm.at[pl.ds(off, GATH_CH)], buf_m.at[sp],
                             sem_m.at[sp])

        @pl.when(jnp.logical_and(i >= 1, i <= niter))
        def _():
            def row(r, rc):
                for j in range(D // 16):
                    sl = pl.ds(j * 16, 16)
                    x = buf_m[sq, r, sl] + buf_a[sq, r, sl] + buf_b[sq, r, sl]
                    buf_m[sq, r, sl] = jnp.maximum(x, 0.0)
                return rc

            lax.fori_loop(0, GATH_CH, row, 0)
            pltpu.async_copy(buf_m.at[sq], out_hbm.at[pl.ds(offq, GATH_CH)],
                             sem_o.at[sq])

    def step(g, carry):
        substep(2 * g, 0, 1)
        substep(2 * g + 1, 1, 0)
        return carry

    lax.fori_loop(0, (GATH_CHUNKS // NW + 2) // 2, step, 0)

    last = niter - 1
    lastslot = lax.rem(last, 2)
    pltpu.make_async_copy(buf_m.at[lastslot],
                          out_hbm.at[pl.ds((wid + last * NW) * GATH_CH,
                                           GATH_CH)],
                          sem_o.at[lastslot]).wait()


def _final(M, AB, Bh, idx_packed):
    mesh = plsc.VectorSubcoreMesh(core_axis_name="c", subcore_axis_name="s")
    return pl.kernel(
        _final_tec,
        out_type=jax.ShapeDtypeStruct((E, D), jnp.float32),
        mesh=mesh,
        scratch_types=[
            pltpu.VMEM((2, GATH_CH, D), jnp.float32),
            pltpu.VMEM((2, GATH_CH, D), jnp.float32),
            pltpu.VMEM((2, GATH_CH, D), jnp.float32),
            pltpu.VMEM((2, 2, GATH_CH), jnp.int32),
            pltpu.SemaphoreType.DMA((2,)),
            pltpu.SemaphoreType.DMA((2,)),
            pltpu.SemaphoreType.DMA((2,)),
            pltpu.SemaphoreType.DMA((2,)),
            pltpu.SemaphoreType.DMA((2,)),
        ],
    )(M, AB, Bh, idx_packed)


@jax.jit
def kernel(nf, ef, edge_index, W_node, W_edge, bias_n, bias_e, W_dense, b_dense):
    src = edge_index[0]
    dst = edge_index[1]
    zS = jnp.zeros((N_PAD, D), jnp.float32)
    zdeg = jnp.zeros((N_PAD,), jnp.float32)
    nf_pad = jnp.concatenate([nf, jnp.zeros((N_PAD - N, D), jnp.float32)], axis=0)

    idx_packed = jnp.stack([dst.reshape(GATH_CHUNKS, GATH_CH),
                            src.reshape(GATH_CHUNKS, GATH_CH)], axis=1)

    nf2p = _nf2(nf_pad, W_node, bias_n)
    S_part, deg_part = _scatter(ef, dst, zS, zdeg)
    M = _m_edges(ef, W_edge, W_dense, b_dense + bias_e)
    AB, Bh = _tables(S_part, deg_part, nf2p, W_edge, W_dense)
    ef2 = _final(M, AB, Bh, idx_packed)
    return (nf2p[:N], ef2)


# speedup vs baseline: 1.3410x; 1.0030x over previous
import functools
import jax
import jax.numpy as jnp
from jax import lax
from jax.experimental import pallas as pl
from jax.experimental.pallas import tpu as pltpu
from jax.experimental.pallas import tpu_sc as plsc

N = 10000
E = 320000
D = 128

NC = 2
NS = 16
NW = NC * NS

E_PER_TILE = E // NW
SCAT_CH = 80
SCAT_ITERS = E_PER_TILE // SCAT_CH
GATH_CH = 128
GATH_CHUNKS = E // GATH_CH
HIGH = jax.lax.Precision.HIGHEST
N_PAD = 10240
N_PER_TILE = N_PAD // NS


def _nf2_body(nf_ref, w_ref, b_ref, o_ref):
    x = jnp.dot(nf_ref[...], w_ref[...], preferred_element_type=jnp.float32,
                precision=HIGH)
    o_ref[...] = jnp.maximum(x + b_ref[...][None, :], 0.0)


def _nf2(nf, W_node, bias_n):
    blk = 2048
    return pl.pallas_call(
        _nf2_body,
        grid=(N_PAD // blk,),
        in_specs=[
            pl.BlockSpec((blk, D), lambda i: (i, 0)),
            pl.BlockSpec((D, D), lambda i: (0, 0)),
            pl.BlockSpec((D,), lambda i: (0,)),
        ],
        out_specs=pl.BlockSpec((blk, D), lambda i: (i, 0)),
        out_shape=jax.ShapeDtypeStruct((N_PAD, D), jnp.float32),
    )(nf, W_node, bias_n)


def _m_body(ef_ref, we_ref, wd_ref, bv_ref, o_ref):
    wc = jnp.dot(we_ref[...], wd_ref[...][:, :D].T,
                 preferred_element_type=jnp.float32, precision=HIGH)
    x = jnp.dot(ef_ref[...], wc, preferred_element_type=jnp.float32)
    o_ref[...] = x + bv_ref[...][None, :]


def _m_edges(ef, W_edge, W_dense, bvec):
    blk = 2560
    return pl.pallas_call(
        _m_body,
        grid=(E // blk,),
        in_specs=[
            pl.BlockSpec((blk, D), lambda i: (i, 0)),
            pl.BlockSpec((D, D), lambda i: (0, 0)),
            pl.BlockSpec((D, 2 * D), lambda i: (0, 0)),
            pl.BlockSpec((D,), lambda i: (0,)),
        ],
        out_specs=pl.BlockSpec((blk, D), lambda i: (i, 0)),
        out_shape=jax.ShapeDtypeStruct((E, D), jnp.float32),
    )(ef, W_edge, W_dense, bvec)


def _tables_body(sp_ref, dp_ref, nf2_ref, we_ref, wd_ref, ab_ref, bh_ref):
    i = pl.program_id(0)
    blk = ab_ref.shape[0]
    S = sp_ref[0] + sp_ref[1]
    deg = dp_ref[0, pl.ds(i * blk, blk)] + dp_ref[1, pl.ds(i * blk, blk)]
    nb = jnp.dot(S, we_ref[...], preferred_element_type=jnp.float32,
                 precision=HIGH)
    nb = nb / jnp.maximum(deg, 1.0)[:, None]
    bh = 0.5 * jnp.dot(nf2_ref[...], wd_ref[...][:, D:].T,
                       preferred_element_type=jnp.float32, precision=HIGH)
    ab = jnp.dot(nb, wd_ref[...][:, :D].T,
                 preferred_element_type=jnp.float32, precision=HIGH) + bh
    ab_ref[...] = ab
    bh_ref[...] = bh


def _tables(S_part, deg_part, nf2, W_edge, W_dense):
    blk = 2048
    return pl.pallas_call(
        _tables_body,
        grid=(N_PAD // blk,),
        in_specs=[
            pl.BlockSpec((NC, blk, D), lambda i: (0, i, 0)),
            pl.BlockSpec((NC, N_PAD), lambda i: (0, 0)),
            pl.BlockSpec((blk, D), lambda i: (i, 0)),
            pl.BlockSpec((D, D), lambda i: (0, 0)),
            pl.BlockSpec((D, 2 * D), lambda i: (0, 0)),
        ],
        out_specs=[
            pl.BlockSpec((blk, D), lambda i: (i, 0)),
            pl.BlockSpec((blk, D), lambda i: (i, 0)),
        ],
        out_shape=[
            jax.ShapeDtypeStruct((N_PAD, D), jnp.float32),
            jax.ShapeDtypeStruct((N_PAD, D), jnp.float32),
        ],
    )(S_part, deg_part, nf2, W_edge, W_dense)


def _scatter_tec(ef_hbm, dst_hbm, zS_hbm, zdeg_hbm, S_out, deg_out,
                 S_acc, deg_acc, ef_buf, idx_buf, ones_buf,
                 sem_li, sem_le, sem_se, sem_sd):
    c = lax.axis_index("c")
    s = lax.axis_index("s")

    pltpu.sync_copy(zS_hbm.at[pl.ds(s * N_PER_TILE, N_PER_TILE)],
                    S_acc.at[pl.ds(s * N_PER_TILE, N_PER_TILE)])

    @pl.when(s == 0)
    def _():
        pltpu.sync_copy(zdeg_hbm, deg_acc)

    for j in range(SCAT_CH // 16):
        ones_buf[pl.ds(j * 16, 16)] = jnp.ones((16,), jnp.float32)

    plsc.subcore_barrier()

    base = (c * NS + s) * E_PER_TILE

    def issue_load(i, slot):
        off = base + i * SCAT_CH
        pltpu.async_copy(dst_hbm.at[pl.ds(off, SCAT_CH)], idx_buf.at[slot],
                         sem_li.at[slot])
        pltpu.async_copy(ef_hbm.at[pl.ds(off, SCAT_CH)], ef_buf.at[slot],
                         sem_le.at[slot])

    issue_load(0, 0)

    def substep(i, sp, sq):
        off = base + i * SCAT_CH

        @pl.when(i >= 1)
        def _():
            pltpu.make_async_copy(ef_buf.at[sq], S_acc.at[idx_buf.at[sq]],
                                  sem_se.at[sq]).wait()
            pltpu.make_async_copy(ones_buf, deg_acc.at[idx_buf.at[sq]],
                                  sem_sd.at[sq]).wait()

        @pl.when(i + 1 < SCAT_ITERS)
        def _():
            issue_load(i + 1, sq)

        @pl.when(i < SCAT_ITERS)
        def _():
            pltpu.make_async_copy(dst_hbm.at[pl.ds(off, SCAT_CH)],
                                  idx_buf.at[sp], sem_li.at[sp]).wait()
            pltpu.make_async_copy(ef_hbm.at[pl.ds(off, SCAT_CH)],
                                  ef_buf.at[sp], sem_le.at[sp]).wait()
            pltpu.async_copy(ef_buf.at[sp], S_acc.at[idx_buf.at[sp]],
                             sem_se.at[sp], add=True)
            pltpu.async_copy(ones_buf, deg_acc.at[idx_buf.at[sp]],
                             sem_sd.at[sp], add=True)

    def step(g, carry):
        substep(2 * g, 0, 1)
        substep(2 * g + 1, 1, 0)
        return carry

    lax.fori_loop(0, (SCAT_ITERS + 1) // 2, step, 0)

    plsc.subcore_barrier()

    pltpu.sync_copy(S_acc.at[pl.ds(s * N_PER_TILE, N_PER_TILE)],
                    S_out.at[c].at[pl.ds(s * N_PER_TILE, N_PER_TILE)])

    @pl.when(s == 0)
    def _():
        pltpu.sync_copy(deg_acc, deg_out.at[c])


def _scatter(ef, dst, zS, zdeg):
    mesh = plsc.VectorSubcoreMesh(core_axis_name="c", subcore_axis_name="s")
    return pl.kernel(
        _scatter_tec,
        out_type=[
            jax.ShapeDtypeStruct((NC, N_PAD, D), jnp.float32),
            jax.ShapeDtypeStruct((NC, N_PAD), jnp.float32),
        ],
        mesh=mesh,
        scratch_types=[
            pltpu.VMEM_SHARED((N_PAD, D), jnp.float32),
            pltpu.VMEM_SHARED((N_PAD,), jnp.float32),
            pltpu.VMEM((2, SCAT_CH, D), jnp.float32),
            pltpu.VMEM((2, SCAT_CH), jnp.int32),
            pltpu.VMEM((SCAT_CH,), jnp.float32),
            pltpu.SemaphoreType.DMA((2,)),
            pltpu.SemaphoreType.DMA((2,)),
            pltpu.SemaphoreType.DMA((2,)),
            pltpu.SemaphoreType.DMA((2,)),
        ],
    )(ef, dst, zS, zdeg)


def _final_tec(m_hbm, ab_hbm, bh_hbm, idx_hbm, out_hbm,
               buf_m, buf_a, buf_b, ibuf,
               sem_i, sem_a, sem_b, sem_m, sem_o):
    c = lax.axis_index("c")
    s = lax.axis_index("s")
    wid = c * NS + s
    niter = jnp.where(wid < GATH_CHUNKS - (GATH_CHUNKS // NW) * NW,
                      GATH_CHUNKS // NW + 1, GATH_CHUNKS // NW)

    def issue_idx(i, slot):
        pltpu.async_copy(idx_hbm.at[wid + i * NW], ibuf.at[slot],
                         sem_i.at[slot])

    issue_idx(0, 0)

    def substep(i, sp, sq):
        off = (wid + i * NW) * GATH_CH
        offq = (wid + (i - 1) * NW) * GATH_CH

        @pl.when(i < niter)
        def _():
            pltpu.make_async_copy(idx_hbm.at[wid + i * NW], ibuf.at[sp],
                                  sem_i.at[sp]).wait()

        @pl.when(jnp.logical_and(i >= 1, i <= niter))
        def _():
            pltpu.make_async_copy(ab_hbm.at[ibuf.at[sq].at[0]], buf_a.at[sq],
                                  sem_a.at[sq]).wait()
            pltpu.make_async_copy(bh_hbm.at[ibuf.at[sq].at[1]], buf_b.at[sq],
                                  sem_b.at[sq]).wait()
            pltpu.make_async_copy(m_hbm.at[pl.ds(offq, GATH_CH)],
                                  buf_m.at[sq], sem_m.at[sq]).wait()

        @pl.when(i + 1 < niter)
        def _():
            issue_idx(i + 1, sq)

        @pl.when(jnp.logical_and(i >= 2, i <= niter))
        def _():
            offp = (wid + (i - 2) * NW) * GATH_CH
            pltpu.make_async_copy(buf_m.at[sp],
                                  out_hbm.at[pl.ds(offp, GATH_CH)],
                                  sem_o.at[sp]).wait()

        @pl.when(i < niter)
        def _():
            pltpu.async_copy(ab_hbm.at[ibuf.at[sp].at[0]], buf_a.at[sp],
                             sem_a.at[sp])
            pltpu.async_copy(bh_hbm.at[ibuf.at[sp].at[1]], buf_b.at[sp],
                             sem_b.at[sp])
            pltpu.async_copy(m_hbm.at[pl.ds(off, GATH_CH)], buf_m.at[sp],
                             sem_m.at[sp])

        @pl.when(jnp.logical_and(i >= 1, i <= niter))
        def _():
            def row(r, rc):
                for j in range(D // 16):
                    sl = pl.ds(j * 16, 16)
                    x = buf_m[sq, r, sl] + buf_a[sq, r, sl] + buf_b[sq, r, sl]
                    buf_m[sq, r, sl] = jnp.maximum(x, 0.0)
                return rc

            lax.fori_loop(0, GATH_CH, row, 0)
            pltpu.async_copy(buf_m.at[sq], out_hbm.at[pl.ds(offq, GATH_CH)],
                             sem_o.at[sq])

    def step(g, carry):
        substep(2 * g, 0, 1)
        substep(2 * g + 1, 1, 0)
        return carry

    lax.fori_loop(0, (GATH_CHUNKS // NW + 2) // 2, step, 0)

    last = niter - 1
    lastslot = lax.rem(last, 2)
    pltpu.make_async_copy(buf_m.at[lastslot],
                          out_hbm.at[pl.ds((wid + last * NW) * GATH_CH,
                                           GATH_CH)],
                          sem_o.at[lastslot]).wait()


def _final(M, AB, Bh, idx_packed):
    mesh = plsc.VectorSubcoreMesh(core_axis_name="c", subcore_axis_name="s")
    return pl.kernel(
        _final_tec,
        out_type=jax.ShapeDtypeStruct((E, D), jnp.float32),
        mesh=mesh,
        scratch_types=[
            pltpu.VMEM((2, GATH_CH, D), jnp.float32),
            pltpu.VMEM((2, GATH_CH, D), jnp.float32),
            pltpu.VMEM((2, GATH_CH, D), jnp.float32),
            pltpu.VMEM((2, 2, GATH_CH), jnp.int32),
            pltpu.SemaphoreType.DMA((2,)),
            pltpu.SemaphoreType.DMA((2,)),
            pltpu.SemaphoreType.DMA((2,)),
            pltpu.SemaphoreType.DMA((2,)),
            pltpu.SemaphoreType.DMA((2,)),
        ],
    )(M, AB, Bh, idx_packed)


@jax.jit
def kernel(nf, ef, edge_index, W_node, W_edge, bias_n, bias_e, W_dense, b_dense):
    src = edge_index[0]
    dst = edge_index[1]
    zS = jnp.zeros((N_PAD, D), jnp.float32)
    zdeg = jnp.zeros((N_PAD,), jnp.float32)
    nf_pad = jnp.concatenate([nf, jnp.zeros((N_PAD - N, D), jnp.float32)], axis=0)

    idx_packed = jnp.stack([dst.reshape(GATH_CHUNKS, GATH_CH),
                            src.reshape(GATH_CHUNKS, GATH_CH)], axis=1)

    nf2p = _nf2(nf_pad, W_node, bias_n)
    S_part, deg_part = _scatter(ef, dst, zS, zdeg)
    M = _m_edges(ef, W_edge, W_dense, b_dense + bias_e)
    AB, Bh = _tables(S_part, deg_part, nf2p, W_edge, W_dense)
    ef2 = _final(M, AB, Bh, idx_packed)
    return (nf2p[:N], ef2)


# speedup vs baseline: 1.3973x; 1.0420x over previous
import functools
import jax
import jax.numpy as jnp
from jax import lax
from jax.experimental import pallas as pl
from jax.experimental.pallas import tpu as pltpu
from jax.experimental.pallas import tpu_sc as plsc

N = 10000
E = 320000
D = 128

NC = 2
NS = 16
NW = NC * NS

E_PER_TILE = E // NW
SCAT_CH = 80
SCAT_ITERS = E_PER_TILE // SCAT_CH
GATH_CH = 128
GATH_CHUNKS = E // GATH_CH
HIGH = jax.lax.Precision.HIGHEST
N_PAD = 10240
N_PER_TILE = N_PAD // NS


def _nf2_body(nf_ref, w_ref, b_ref, o_ref):
    x = jnp.dot(nf_ref[...], w_ref[...], preferred_element_type=jnp.float32,
                precision=HIGH)
    o_ref[...] = jnp.maximum(x + b_ref[...][None, :], 0.0)


def _nf2(nf, W_node, bias_n):
    blk = 2048
    return pl.pallas_call(
        _nf2_body,
        grid=(N_PAD // blk,),
        in_specs=[
            pl.BlockSpec((blk, D), lambda i: (i, 0)),
            pl.BlockSpec((D, D), lambda i: (0, 0)),
            pl.BlockSpec((D,), lambda i: (0,)),
        ],
        out_specs=pl.BlockSpec((blk, D), lambda i: (i, 0)),
        out_shape=jax.ShapeDtypeStruct((N_PAD, D), jnp.float32),
    )(nf, W_node, bias_n)


def _m_body(ef_ref, we_ref, wd_ref, bv_ref, o_ref):
    wc = jnp.dot(we_ref[...], wd_ref[...][:, :D].T,
                 preferred_element_type=jnp.float32, precision=HIGH)
    x = jnp.dot(ef_ref[...], wc, preferred_element_type=jnp.float32)
    o_ref[...] = x + bv_ref[...][None, :]


def _m_edges(ef, W_edge, W_dense, bvec):
    blk = 6400
    return pl.pallas_call(
        _m_body,
        grid=(E // blk,),
        in_specs=[
            pl.BlockSpec((blk, D), lambda i: (i, 0)),
            pl.BlockSpec((D, D), lambda i: (0, 0)),
            pl.BlockSpec((D, 2 * D), lambda i: (0, 0)),
            pl.BlockSpec((D,), lambda i: (0,)),
        ],
        out_specs=pl.BlockSpec((blk, D), lambda i: (i, 0)),
        out_shape=jax.ShapeDtypeStruct((E, D), jnp.float32),
    )(ef, W_edge, W_dense, bvec)


def _tables_body(sp_ref, dp_ref, nf2_ref, we_ref, wd_ref, ab_ref, bh_ref):
    i = pl.program_id(0)
    blk = ab_ref.shape[0]
    S = sp_ref[0] + sp_ref[1]
    deg = dp_ref[0, pl.ds(i * blk, blk)] + dp_ref[1, pl.ds(i * blk, blk)]
    nb = jnp.dot(S, we_ref[...], preferred_element_type=jnp.float32,
                 precision=HIGH)
    nb = nb / jnp.maximum(deg, 1.0)[:, None]
    bh = 0.5 * jnp.dot(nf2_ref[...], wd_ref[...][:, D:].T,
                       preferred_element_type=jnp.float32, precision=HIGH)
    ab = jnp.dot(nb, wd_ref[...][:, :D].T,
                 preferred_element_type=jnp.float32, precision=HIGH) + bh
    ab_ref[...] = ab
    bh_ref[...] = bh


def _tables(S_part, deg_part, nf2, W_edge, W_dense):
    blk = 2048
    return pl.pallas_call(
        _tables_body,
        grid=(N_PAD // blk,),
        in_specs=[
            pl.BlockSpec((NC, blk, D), lambda i: (0, i, 0)),
            pl.BlockSpec((NC, N_PAD), lambda i: (0, 0)),
            pl.BlockSpec((blk, D), lambda i: (i, 0)),
            pl.BlockSpec((D, D), lambda i: (0, 0)),
            pl.BlockSpec((D, 2 * D), lambda i: (0, 0)),
        ],
        out_specs=[
            pl.BlockSpec((blk, D), lambda i: (i, 0)),
            pl.BlockSpec((blk, D), lambda i: (i, 0)),
        ],
        out_shape=[
            jax.ShapeDtypeStruct((N_PAD, D), jnp.float32),
            jax.ShapeDtypeStruct((N_PAD, D), jnp.float32),
        ],
    )(S_part, deg_part, nf2, W_edge, W_dense)


def _scatter_tec(ef_hbm, dst_hbm, zS_hbm, zdeg_hbm, S_out, deg_out,
                 S_acc, deg_acc, ef_buf, idx_buf, ones_buf,
                 sem_li, sem_le, sem_se, sem_sd):
    c = lax.axis_index("c")
    s = lax.axis_index("s")

    pltpu.sync_copy(zS_hbm.at[pl.ds(s * N_PER_TILE, N_PER_TILE)],
                    S_acc.at[pl.ds(s * N_PER_TILE, N_PER_TILE)])

    @pl.when(s == 0)
    def _():
        pltpu.sync_copy(zdeg_hbm, deg_acc)

    for j in range(SCAT_CH // 16):
        ones_buf[pl.ds(j * 16, 16)] = jnp.ones((16,), jnp.float32)

    plsc.subcore_barrier()

    base = (c * NS + s) * E_PER_TILE

    def issue_load(i, slot):
        off = base + i * SCAT_CH
        pltpu.async_copy(dst_hbm.at[pl.ds(off, SCAT_CH)], idx_buf.at[slot],
                         sem_li.at[slot])
        pltpu.async_copy(ef_hbm.at[pl.ds(off, SCAT_CH)], ef_buf.at[slot],
                         sem_le.at[slot])

    issue_load(0, 0)

    def substep(i, sp, sq):
        off = base + i * SCAT_CH

        @pl.when(i >= 1)
        def _():
            pltpu.make_async_copy(ef_buf.at[sq], S_acc.at[idx_buf.at[sq]],
                                  sem_se.at[sq]).wait()
            pltpu.make_async_copy(ones_buf, deg_acc.at[idx_buf.at[sq]],
                                  sem_sd.at[sq]).wait()

        @pl.when(i + 1 < SCAT_ITERS)
        def _():
            issue_load(i + 1, sq)

        @pl.when(i < SCAT_ITERS)
        def _():
            pltpu.make_async_copy(dst_hbm.at[pl.ds(off, SCAT_CH)],
                                  idx_buf.at[sp], sem_li.at[sp]).wait()
            pltpu.make_async_copy(ef_hbm.at[pl.ds(off, SCAT_CH)],
                                  ef_buf.at[sp], sem_le.at[sp]).wait()
            pltpu.async_copy(ef_buf.at[sp], S_acc.at[idx_buf.at[sp]],
                             sem_se.at[sp], add=True)
            pltpu.async_copy(ones_buf, deg_acc.at[idx_buf.at[sp]],
                             sem_sd.at[sp], add=True)

    def step(g, carry):
        substep(2 * g, 0, 1)
        substep(2 * g + 1, 1, 0)
        return carry

    lax.fori_loop(0, (SCAT_ITERS + 1) // 2, step, 0)

    plsc.subcore_barrier()

    pltpu.sync_copy(S_acc.at[pl.ds(s * N_PER_TILE, N_PER_TILE)],
                    S_out.at[c].at[pl.ds(s * N_PER_TILE, N_PER_TILE)])

    @pl.when(s == 0)
    def _():
        pltpu.sync_copy(deg_acc, deg_out.at[c])


def _scatter(ef, dst, zS, zdeg):
    mesh = plsc.VectorSubcoreMesh(core_axis_name="c", subcore_axis_name="s")
    return pl.kernel(
        _scatter_tec,
        out_type=[
            jax.ShapeDtypeStruct((NC, N_PAD, D), jnp.float32),
            jax.ShapeDtypeStruct((NC, N_PAD), jnp.float32),
        ],
        mesh=mesh,
        scratch_types=[
            pltpu.VMEM_SHARED((N_PAD, D), jnp.float32),
            pltpu.VMEM_SHARED((N_PAD,), jnp.float32),
            pltpu.VMEM((2, SCAT_CH, D), jnp.float32),
            pltpu.VMEM((2, SCAT_CH), jnp.int32),
            pltpu.VMEM((SCAT_CH,), jnp.float32),
            pltpu.SemaphoreType.DMA((2,)),
            pltpu.SemaphoreType.DMA((2,)),
            pltpu.SemaphoreType.DMA((2,)),
            pltpu.SemaphoreType.DMA((2,)),
        ],
    )(ef, dst, zS, zdeg)


def _final_tec(m_hbm, ab_hbm, bh_hbm, idx_hbm, out_hbm,
               buf_m, buf_a, buf_b, ibuf,
               sem_i, sem_a, sem_b, sem_m, sem_o):
    c = lax.axis_index("c")
    s = lax.axis_index("s")
    wid = c * NS + s
    niter = jnp.where(wid < GATH_CHUNKS - (GATH_CHUNKS // NW) * NW,
                      GATH_CHUNKS // NW + 1, GATH_CHUNKS // NW)

    def issue_idx(i, slot):
        pltpu.async_copy(idx_hbm.at[wid + i * NW], ibuf.at[slot],
                         sem_i.at[slot])

    issue_idx(0, 0)

    def substep(i, sp, sq):
        off = (wid + i * NW) * GATH_CH
        offq = (wid + (i - 1) * NW) * GATH_CH

        @pl.when(i < niter)
        def _():
            pltpu.make_async_copy(idx_hbm.at[wid + i * NW], ibuf.at[sp],
                                  sem_i.at[sp]).wait()

        @pl.when(jnp.logical_and(i >= 1, i <= niter))
        def _():
            pltpu.make_async_copy(ab_hbm.at[ibuf.at[sq].at[0]], buf_a.at[sq],
                                  sem_a.at[sq]).wait()
            pltpu.make_async_copy(bh_hbm.at[ibuf.at[sq].at[1]], buf_b.at[sq],
                                  sem_b.at[sq]).wait()
            pltpu.make_async_copy(m_hbm.at[pl.ds(offq, GATH_CH)],
                                  buf_m.at[sq], sem_m.at[sq]).wait()

        @pl.when(i + 1 < niter)
        def _():
            issue_idx(i + 1, sq)

        @pl.when(jnp.logical_and(i >= 2, i <= niter))
        def _():
            offp = (wid + (i - 2) * NW) * GATH_CH
            pltpu.make_async_copy(buf_m.at[sp],
                                  out_hbm.at[pl.ds(offp, GATH_CH)],
                                  sem_o.at[sp]).wait()

        @pl.when(i < niter)
        def _():
            pltpu.async_copy(ab_hbm.at[ibuf.at[sp].at[0]], buf_a.at[sp],
                             sem_a.at[sp])
            pltpu.async_copy(bh_hbm.at[ibuf.at[sp].at[1]], buf_b.at[sp],
                             sem_b.at[sp])
            pltpu.async_copy(m_hbm.at[pl.ds(off, GATH_CH)], buf_m.at[sp],
                             sem_m.at[sp])

        @pl.when(jnp.logical_and(i >= 1, i <= niter))
        def _():
            def row(r, rc):
                for j in range(D // 16):
                    sl = pl.ds(j * 16, 16)
                    x = buf_m[sq, r, sl] + buf_a[sq, r, sl] + buf_b[sq, r, sl]
                    buf_m[sq, r, sl] = jnp.maximum(x, 0.0)
                return rc

            lax.fori_loop(0, GATH_CH, row, 0)
            pltpu.async_copy(buf_m.at[sq], out_hbm.at[pl.ds(offq, GATH_CH)],
                             sem_o.at[sq])

    def step(g, carry):
        substep(2 * g, 0, 1)
        substep(2 * g + 1, 1, 0)
        return carry

    lax.fori_loop(0, (GATH_CHUNKS // NW + 2) // 2, step, 0)

    last = niter - 1
    lastslot = lax.rem(last, 2)
    pltpu.make_async_copy(buf_m.at[lastslot],
                          out_hbm.at[pl.ds((wid + last * NW) * GATH_CH,
                                           GATH_CH)],
                          sem_o.at[lastslot]).wait()


def _final(M, AB, Bh, idx_packed):
    mesh = plsc.VectorSubcoreMesh(core_axis_name="c", subcore_axis_name="s")
    return pl.kernel(
        _final_tec,
        out_type=jax.ShapeDtypeStruct((E, D), jnp.float32),
        mesh=mesh,
        scratch_types=[
            pltpu.VMEM((2, GATH_CH, D), jnp.float32),
            pltpu.VMEM((2, GATH_CH, D), jnp.float32),
            pltpu.VMEM((2, GATH_CH, D), jnp.float32),
            pltpu.VMEM((2, 2, GATH_CH), jnp.int32),
            pltpu.SemaphoreType.DMA((2,)),
            pltpu.SemaphoreType.DMA((2,)),
            pltpu.SemaphoreType.DMA((2,)),
            pltpu.SemaphoreType.DMA((2,)),
            pltpu.SemaphoreType.DMA((2,)),
        ],
    )(M, AB, Bh, idx_packed)


@jax.jit
def kernel(nf, ef, edge_index, W_node, W_edge, bias_n, bias_e, W_dense, b_dense):
    src = edge_index[0]
    dst = edge_index[1]
    zS = jnp.zeros((N_PAD, D), jnp.float32)
    zdeg = jnp.zeros((N_PAD,), jnp.float32)
    nf_pad = jnp.concatenate([nf, jnp.zeros((N_PAD - N, D), jnp.float32)], axis=0)

    idx_packed = jnp.stack([dst.reshape(GATH_CHUNKS, GATH_CH),
                            src.reshape(GATH_CHUNKS, GATH_CH)], axis=1)

    nf2p = _nf2(nf_pad, W_node, bias_n)
    S_part, deg_part = _scatter(ef, dst, zS, zdeg)
    M = _m_edges(ef, W_edge, W_dense, b_dense + bias_e)
    AB, Bh = _tables(S_part, deg_part, nf2p, W_edge, W_dense)
    ef2 = _final(M, AB, Bh, idx_packed)
    return (nf2p[:N], ef2)
